# baseline (speedup 1.0000x reference)
import jax
import jax.numpy as jnp
from jax.experimental import pallas as pl
from jax.experimental.pallas import tpu as pltpu

N = 10000
E = 2000
F_IN = 256
HID = 256
F_OUT = 256

TN = 400
NS = 5
RS = TN // NS

_INV_SCALE = 1.0 / 16.0
_DN_T = (((0,), (0,)), ((), ()))
_DN_RT = (((1,), (1,)), ((), ()))


def _hf_kernel(x_ref, h0, h1, h2, h3, h4, o_ref):
    i = pl.program_id(0)
    tot = None
    for j, h in enumerate((h0, h1, h2, h3, h4)):
        part = jax.lax.dot_general(
            x_ref[pl.ds(j * RS, RS), :], h[...], _DN_T,
            preferred_element_type=jnp.float32)
        tot = part if tot is None else tot + part

    @pl.when(i == 0)
    def _():
        o_ref[...] = tot

    @pl.when(i > 0)
    def _():
        o_ref[...] = o_ref[...] + tot


def _edge_attn_kernel(hft_ref, wq_ref, bq_ref, wk_ref, bk_ref, wv_ref,
                      bv_ref, wnk_ref, bnk_ref, att_ref, kn_ref):
    hf = hft_ref[...].T
    q = jnp.dot(hf, wq_ref[...], preferred_element_type=jnp.float32) + bq_ref[...]
    k = jnp.dot(hf, wk_ref[...], preferred_element_type=jnp.float32) + bk_ref[...]
    v = jnp.dot(hf, wv_ref[...], preferred_element_type=jnp.float32) + bv_ref[...]
    s = jax.lax.dot_general(
        q, k, _DN_RT, preferred_element_type=jnp.float32) * _INV_SCALE
    m = jnp.max(s, axis=-1, keepdims=True)
    e = jnp.exp(s - m)
    w = e / jnp.sum(e, axis=-1, keepdims=True)
    att = jnp.dot(w, v, preferred_element_type=jnp.float32)
    att_ref[...] = att
    kn_ref[...] = jnp.dot(att, wnk_ref[...],
                          preferred_element_type=jnp.float32) + bnk_ref[...]


def _node_kernel(x_ref, h0, h1, h2, h3, h4, att_ref, kn_ref, wnq_ref,
                 bnq_ref, wt_ref, bt_ref, o_ref):
    qn = jnp.dot(x_ref[...], wnq_ref[...],
                 preferred_element_type=jnp.float32) + bnq_ref[...]
    s = jax.lax.dot_general(
        qn, kn_ref[...], _DN_RT, preferred_element_type=jnp.float32) * _INV_SCALE
    m = jnp.max(s, axis=-1, keepdims=True)
    e = jnp.exp(s - m)
    w = e / jnp.sum(e, axis=-1, keepdims=True)
    for j, h in enumerate((h0, h1, h2, h3, h4)):
        h_att = h[...] * w[j * RS:(j + 1) * RS, :]
        agg = jnp.dot(h_att, att_ref[...], preferred_element_type=jnp.float32)
        o_ref[pl.ds(j * RS, RS), :] = jnp.dot(
            agg, wt_ref[...], preferred_element_type=jnp.float32) + bt_ref[...]


@jax.jit
def kernel(X, H_norm, Wq, bq, Wk, bk, Wv, bv, Wnq, bnq, Wnk, bnk, Wt, bt):
    f32 = jnp.float32
    bq2, bk2, bv2 = bq.reshape(1, HID), bk.reshape(1, HID), bv.reshape(1, HID)
    bnk2 = bnk.reshape(1, HID)
    bnq2, bt2 = bnq.reshape(1, HID), bt.reshape(1, F_OUT)

    full = lambda shape: pl.BlockSpec(shape, lambda *_: tuple(0 for _s in shape))

    def h_strip(j):
        return pl.BlockSpec((RS, E), lambda i, j=j: (NS * i + j, 0))

    hft = pl.pallas_call(
        _hf_kernel,
        grid=(N // TN,),
        in_specs=[pl.BlockSpec((TN, F_IN), lambda i: (i, 0))]
        + [h_strip(j) for j in range(NS)],
        out_specs=full((F_IN, E)),
        out_shape=jax.ShapeDtypeStruct((F_IN, E), f32),
        compiler_params=pltpu.CompilerParams(
            dimension_semantics=("arbitrary",)),
    )(X, H_norm, H_norm, H_norm, H_norm, H_norm)

    attended, kn = pl.pallas_call(
        _edge_attn_kernel,
        in_specs=[
            full((F_IN, E)),
            full((F_IN, HID)), full((1, HID)),
            full((F_IN, HID)), full((1, HID)),
            full((F_IN, HID)), full((1, HID)),
            full((HID, HID)), full((1, HID)),
        ],
        out_specs=[full((E, HID)), full((E, HID))],
        out_shape=[
            jax.ShapeDtypeStruct((E, HID), f32),
            jax.ShapeDtypeStruct((E, HID), f32),
        ],
    )(hft, Wq, bq2, Wk, bk2, Wv, bv2, Wnk, bnk2)

    out = pl.pallas_call(
        _node_kernel,
        grid=(N // TN,),
        in_specs=[pl.BlockSpec((TN, F_IN), lambda i: (i, 0))]
        + [h_strip(j) for j in range(NS)]
        + [
            full((E, HID)),
            full((E, HID)),
            full((F_IN, HID)), full((1, HID)),
            full((HID, F_OUT)), full((1, F_OUT)),
        ],
        out_specs=pl.BlockSpec((TN, F_OUT), lambda i: (i, 0)),
        out_shape=jax.ShapeDtypeStruct((N, F_OUT), f32),
        compiler_params=pltpu.CompilerParams(
            dimension_semantics=("parallel",)),
    )(X, H_norm, H_norm, H_norm, H_norm, H_norm, attended, kn,
      Wnq, bnq2, Wt, bt2)

    return out


# speedup vs baseline: 1.1354x; 1.1354x over previous
import jax
import jax.numpy as jnp
from jax.experimental import pallas as pl
from jax.experimental.pallas import tpu as pltpu

N = 10000
E = 2000
F_IN = 256
HID = 256
F_OUT = 256

TN = 400

_INV_SCALE = 1.0 / 16.0
_DN_T = (((0,), (0,)), ((), ()))
_DN_RT = (((1,), (1,)), ((), ()))


def _hf_kernel(x_ref, h_ref, o_ref, acc_ref):
    i = pl.program_id(0)
    part = jax.lax.dot_general(
        x_ref[...], h_ref[...], _DN_T, preferred_element_type=jnp.float32)

    @pl.when(i == 0)
    def _():
        acc_ref[...] = part

    @pl.when(i > 0)
    def _():
        acc_ref[...] = acc_ref[...] + part

    @pl.when(i == N // TN - 1)
    def _():
        o_ref[...] = acc_ref[...]


def _edge_attn_kernel(hft_ref, wq_ref, bq_ref, wk_ref, bk_ref, wv_ref,
                      bv_ref, wnk_ref, bnk_ref, att_ref, kn_ref):
    hf = hft_ref[...].T
    q = jnp.dot(hf, wq_ref[...], preferred_element_type=jnp.float32) + bq_ref[...]
    k = jnp.dot(hf, wk_ref[...], preferred_element_type=jnp.float32) + bk_ref[...]
    v = jnp.dot(hf, wv_ref[...], preferred_element_type=jnp.float32) + bv_ref[...]
    s = jax.lax.dot_general(
        q, k, _DN_RT, preferred_element_type=jnp.float32) * _INV_SCALE
    m = jnp.max(s, axis=-1, keepdims=True)
    e = jnp.exp(s - m)
    w = e / jnp.sum(e, axis=-1, keepdims=True)
    att = jnp.dot(w, v, preferred_element_type=jnp.float32)
    att_ref[...] = att
    kn_ref[...] = jnp.dot(att, wnk_ref[...],
                          preferred_element_type=jnp.float32) + bnk_ref[...]


def _node_kernel(x_ref, h_ref, att_ref, kn_ref, wnq_ref,
                 bnq_ref, wt_ref, bt_ref, o_ref):
    qn = jnp.dot(x_ref[...], wnq_ref[...],
                 preferred_element_type=jnp.float32) + bnq_ref[...]
    s = jax.lax.dot_general(
        qn, kn_ref[...], _DN_RT, preferred_element_type=jnp.float32) * _INV_SCALE
    m = jnp.max(s, axis=-1, keepdims=True)
    e = jnp.exp(s - m)
    w = e / jnp.sum(e, axis=-1, keepdims=True)
    h_att = h_ref[...] * w
    agg = jnp.dot(h_att, att_ref[...], preferred_element_type=jnp.float32)
    o_ref[...] = jnp.dot(agg, wt_ref[...],
                         preferred_element_type=jnp.float32) + bt_ref[...]


@jax.jit
def kernel(X, H_norm, Wq, bq, Wk, bk, Wv, bv, Wnq, bnq, Wnk, bnk, Wt, bt):
    f32 = jnp.float32
    bq2, bk2, bv2 = bq.reshape(1, HID), bk.reshape(1, HID), bv.reshape(1, HID)
    bnk2 = bnk.reshape(1, HID)
    bnq2, bt2 = bnq.reshape(1, HID), bt.reshape(1, F_OUT)

    full = lambda shape: pl.BlockSpec(shape, lambda *_: tuple(0 for _s in shape))

    hft = pl.pallas_call(
        _hf_kernel,
        grid=(N // TN,),
        in_specs=[
            pl.BlockSpec((TN, F_IN), lambda i: (i, 0)),
            pl.BlockSpec((TN, E), lambda i: (i, 0)),
        ],
        out_specs=pl.BlockSpec((F_IN, E), lambda i: (0, 0)),
        out_shape=jax.ShapeDtypeStruct((F_IN, E), f32),
        scratch_shapes=[pltpu.VMEM((F_IN, E), f32)],
        compiler_params=pltpu.CompilerParams(
            dimension_semantics=("parallel",)),
    )(X, H_norm)

    attended, kn = pl.pallas_call(
        _edge_attn_kernel,
        in_specs=[
            full((F_IN, E)),
            full((F_IN, HID)), full((1, HID)),
            full((F_IN, HID)), full((1, HID)),
            full((F_IN, HID)), full((1, HID)),
            full((HID, HID)), full((1, HID)),
        ],
        out_specs=[full((E, HID)), full((E, HID))],
        out_shape=[
            jax.ShapeDtypeStruct((E, HID), f32),
            jax.ShapeDtypeStruct((E, HID), f32),
        ],
    )(hft, Wq, bq2, Wk, bk2, Wv, bv2, Wnk, bnk2)

    out = pl.pallas_call(
        _node_kernel,
        grid=(N // TN,),
        in_specs=[
            pl.BlockSpec((TN, F_IN), lambda i: (i, 0)),
            pl.BlockSpec((TN, E), lambda i: (i, 0)),
            full((E, HID)),
            full((E, HID)),
            full((F_IN, HID)), full((1, HID)),
            full((HID, F_OUT)), full((1, F_OUT)),
        ],
        out_specs=pl.BlockSpec((TN, F_OUT), lambda i: (i, 0)),
        out_shape=jax.ShapeDtypeStruct((N, F_OUT), f32),
        compiler_params=pltpu.CompilerParams(
            dimension_semantics=("parallel",)),
    )(X, H_norm, attended, kn, Wnq, bnq2, Wt, bt2)

    return out


# speedup vs baseline: 1.1758x; 1.0356x over previous
import jax
import jax.numpy as jnp
from jax.experimental import pallas as pl
from jax.experimental.pallas import tpu as pltpu

N = 10000
E = 2000
F_IN = 256
HID = 256
F_OUT = 256

TN = 400
NA = N // TN
NC = N // TN
STEPS = NA + 1 + NC

_INV_SCALE = 1.0 / 16.0
_DN_T = (((0,), (0,)), ((), ()))
_DN_RT = (((1,), (1,)), ((), ()))


def _fused_kernel(x_ref, h_ref, wq_ref, bq_ref, wk_ref, bk_ref, wv_ref,
                  bv_ref, wnk_ref, bnk_ref, wnq_ref, bnq_ref, wt_ref, bt_ref,
                  o_ref, hft_s, att_s, kn_s):
    i = pl.program_id(0)

    @pl.when(i == 0)
    def _():
        hft_s[...] = jax.lax.dot_general(
            x_ref[...], h_ref[...], _DN_T, preferred_element_type=jnp.float32)

    @pl.when(jnp.logical_and(i > 0, i < NA))
    def _():
        hft_s[...] = hft_s[...] + jax.lax.dot_general(
            x_ref[...], h_ref[...], _DN_T, preferred_element_type=jnp.float32)

    @pl.when(i == NA)
    def _():
        hf = hft_s[...].T
        q = jnp.dot(hf, wq_ref[...],
                    preferred_element_type=jnp.float32) + bq_ref[...]
        k = jnp.dot(hf, wk_ref[...],
                    preferred_element_type=jnp.float32) + bk_ref[...]
        v = jnp.dot(hf, wv_ref[...],
                    preferred_element_type=jnp.float32) + bv_ref[...]
        s = jax.lax.dot_general(
            q, k, _DN_RT, preferred_element_type=jnp.float32) * _INV_SCALE
        m = jnp.max(s, axis=-1, keepdims=True)
        e = jnp.exp(s - m)
        w = e / jnp.sum(e, axis=-1, keepdims=True)
        att = jnp.dot(w, v, preferred_element_type=jnp.float32)
        att_s[...] = att
        kn_s[...] = jnp.dot(att, wnk_ref[...],
                            preferred_element_type=jnp.float32) + bnk_ref[...]

    @pl.when(i > NA)
    def _():
        qn = jnp.dot(x_ref[...], wnq_ref[...],
                     preferred_element_type=jnp.float32) + bnq_ref[...]
        s = jax.lax.dot_general(
            qn, kn_s[...], _DN_RT,
            preferred_element_type=jnp.float32) * _INV_SCALE
        m = jnp.max(s, axis=-1, keepdims=True)
        e = jnp.exp(s - m)
        w = e / jnp.sum(e, axis=-1, keepdims=True)
        h_att = h_ref[...] * w
        agg = jnp.dot(h_att, att_s[...], preferred_element_type=jnp.float32)
        o_ref[...] = jnp.dot(agg, wt_ref[...],
                             preferred_element_type=jnp.float32) + bt_ref[...]


@jax.jit
def kernel(X, H_norm, Wq, bq, Wk, bk, Wv, bv, Wnq, bnq, Wnk, bnk, Wt, bt):
    f32 = jnp.float32
    bq2, bk2, bv2 = bq.reshape(1, HID), bk.reshape(1, HID), bv.reshape(1, HID)
    bnk2 = bnk.reshape(1, HID)
    bnq2, bt2 = bnq.reshape(1, HID), bt.reshape(1, F_OUT)

    full = lambda shape: pl.BlockSpec(shape, lambda *_: tuple(0 for _s in shape))

    def row_idx(i):
        return jnp.where(i < NA, i, jnp.maximum(i - (NA + 1), 0))

    out = pl.pallas_call(
        _fused_kernel,
        grid=(STEPS,),
        in_specs=[
            pl.BlockSpec((TN, F_IN), lambda i: (row_idx(i), 0)),
            pl.BlockSpec((TN, E), lambda i: (row_idx(i), 0)),
            full((F_IN, HID)), full((1, HID)),
            full((F_IN, HID)), full((1, HID)),
            full((F_IN, HID)), full((1, HID)),
            full((HID, HID)), full((1, HID)),
            full((F_IN, HID)), full((1, HID)),
            full((HID, F_OUT)), full((1, F_OUT)),
        ],
        out_specs=pl.BlockSpec((TN, F_OUT), lambda i: (row_idx(i), 0)),
        out_shape=jax.ShapeDtypeStruct((N, F_OUT), f32),
        scratch_shapes=[
            pltpu.VMEM((F_IN, E), f32),
            pltpu.VMEM((E, HID), f32),
            pltpu.VMEM((E, HID), f32),
        ],
        compiler_params=pltpu.CompilerParams(
            dimension_semantics=("arbitrary",)),
    )(X, H_norm, Wq, bq2, Wk, bk2, Wv, bv2, Wnk, bnk2, Wnq, bnq2, Wt, bt2)

    return out


# speedup vs baseline: 1.2921x; 1.0988x over previous
import jax
import jax.numpy as jnp
from jax.experimental import pallas as pl
from jax.experimental.pallas import tpu as pltpu

N = 10000
E = 2000
F_IN = 256
HID = 256
F_OUT = 256

TN = 1000
NA = N // TN
NC = N // TN
STEPS = NA + 1 + NC

_INV_SCALE = 1.0 / 16.0
_DN_T = (((0,), (0,)), ((), ()))
_DN_RT = (((1,), (1,)), ((), ()))


def _fused_kernel(x_ref, h_ref, wq_ref, bq_ref, wk_ref, bk_ref, wv_ref,
                  bv_ref, wnk_ref, bnk_ref, wnq_ref, bnq_ref, wt_ref, bt_ref,
                  o_ref, hft_s, att_s, kn_s):
    i = pl.program_id(0)

    @pl.when(i == 0)
    def _():
        hft_s[...] = jax.lax.dot_general(
            x_ref[...], h_ref[...], _DN_T, preferred_element_type=jnp.float32)

    @pl.when(jnp.logical_and(i > 0, i < NA))
    def _():
        hft_s[...] = hft_s[...] + jax.lax.dot_general(
            x_ref[...], h_ref[...], _DN_T, preferred_element_type=jnp.float32)

    @pl.when(i == NA)
    def _():
        hf = hft_s[...].T
        q = jnp.dot(hf, wq_ref[...],
                    preferred_element_type=jnp.float32) + bq_ref[...]
        k = jnp.dot(hf, wk_ref[...],
                    preferred_element_type=jnp.float32) + bk_ref[...]
        v = jnp.dot(hf, wv_ref[...],
                    preferred_element_type=jnp.float32) + bv_ref[...]
        s = jax.lax.dot_general(
            q, k, _DN_RT, preferred_element_type=jnp.float32) * _INV_SCALE
        m = jnp.max(s, axis=-1, keepdims=True)
        e = jnp.exp(s - m)
        w = e / jnp.sum(e, axis=-1, keepdims=True)
        att = jnp.dot(w, v, preferred_element_type=jnp.float32)
        att_s[...] = att
        kn_s[...] = jnp.dot(att, wnk_ref[...],
                            preferred_element_type=jnp.float32) + bnk_ref[...]

    @pl.when(i > NA)
    def _():
        qn = jnp.dot(x_ref[...], wnq_ref[...],
                     preferred_element_type=jnp.float32) + bnq_ref[...]
        s = jax.lax.dot_general(
            qn, kn_s[...], _DN_RT,
            preferred_element_type=jnp.float32) * _INV_SCALE
        m = jnp.max(s, axis=-1, keepdims=True)
        e = jnp.exp(s - m)
        w = e / jnp.sum(e, axis=-1, keepdims=True)
        h_att = h_ref[...] * w
        agg = jnp.dot(h_att, att_s[...], preferred_element_type=jnp.float32)
        o_ref[...] = jnp.dot(agg, wt_ref[...],
                             preferred_element_type=jnp.float32) + bt_ref[...]


@jax.jit
def kernel(X, H_norm, Wq, bq, Wk, bk, Wv, bv, Wnq, bnq, Wnk, bnk, Wt, bt):
    f32 = jnp.float32
    bq2, bk2, bv2 = bq.reshape(1, HID), bk.reshape(1, HID), bv.reshape(1, HID)
    bnk2 = bnk.reshape(1, HID)
    bnq2, bt2 = bnq.reshape(1, HID), bt.reshape(1, F_OUT)

    full = lambda shape: pl.BlockSpec(shape, lambda *_: tuple(0 for _s in shape))

    def row_idx(i):
        return jnp.where(i < NA, i, jnp.maximum(i - (NA + 1), 0))

    out = pl.pallas_call(
        _fused_kernel,
        grid=(STEPS,),
        in_specs=[
            pl.BlockSpec((TN, F_IN), lambda i: (row_idx(i), 0)),
            pl.BlockSpec((TN, E), lambda i: (row_idx(i), 0)),
            full((F_IN, HID)), full((1, HID)),
            full((F_IN, HID)), full((1, HID)),
            full((F_IN, HID)), full((1, HID)),
            full((HID, HID)), full((1, HID)),
            full((F_IN, HID)), full((1, HID)),
            full((HID, F_OUT)), full((1, F_OUT)),
        ],
        out_specs=pl.BlockSpec((TN, F_OUT), lambda i: (row_idx(i), 0)),
        out_shape=jax.ShapeDtypeStruct((N, F_OUT), f32),
        scratch_shapes=[
            pltpu.VMEM((F_IN, E), f32),
            pltpu.VMEM((E, HID), f32),
            pltpu.VMEM((E, HID), f32),
        ],
        compiler_params=pltpu.CompilerParams(
            dimension_semantics=("arbitrary",)),
    )(X, H_norm, Wq, bq2, Wk, bk2, Wv, bv2, Wnk, bnk2, Wnq, bnq2, Wt, bt2)

    return out


# speedup vs baseline: 1.3519x; 1.0463x over previous
import jax
import jax.numpy as jnp
from jax.experimental import pallas as pl
from jax.experimental.pallas import tpu as pltpu

N = 10000
E = 2000
F_IN = 256
HID = 256
F_OUT = 256

TN = 1000
NA = N // TN
NC = N // TN
STEPS = NA + 1 + NC

_INV_SCALE = 1.0 / 16.0
_DN_T = (((0,), (0,)), ((), ()))
_DN_RT = (((1,), (1,)), ((), ()))


def _fused_kernel(x_ref, h_ref, wq_ref, bq_ref, wk_ref, bk_ref, wv_ref,
                  bv_ref, wnk_ref, bnk_ref, wnq_ref, bnq_ref, wt_ref, bt_ref,
                  o_ref, hft_s, att_s, kn_s):
    i = pl.program_id(0)

    @pl.when(i == 0)
    def _():
        hft_s[...] = jax.lax.dot_general(
            x_ref[...], h_ref[...], _DN_T, preferred_element_type=jnp.float32)

    @pl.when(jnp.logical_and(i > 0, i < NA))
    def _():
        hft_s[...] = hft_s[...] + jax.lax.dot_general(
            x_ref[...], h_ref[...], _DN_T, preferred_element_type=jnp.float32)

    @pl.when(i == NA)
    def _():
        hf = hft_s[...].T
        q = jnp.dot(hf, wq_ref[...],
                    preferred_element_type=jnp.float32) + bq_ref[...]
        k = jnp.dot(hf, wk_ref[...],
                    preferred_element_type=jnp.float32) + bk_ref[...]
        v = jnp.dot(hf, wv_ref[...],
                    preferred_element_type=jnp.float32) + bv_ref[...]
        s = jax.lax.dot_general(
            q, k, _DN_RT, preferred_element_type=jnp.float32) * _INV_SCALE
        m = jnp.max(s, axis=-1, keepdims=True)
        e = jnp.exp(s - m)
        w = e / jnp.sum(e, axis=-1, keepdims=True)
        att = jnp.dot(w, v, preferred_element_type=jnp.float32)
        att_s[...] = att
        kn_s[...] = jnp.dot(att, wnk_ref[...],
                            preferred_element_type=jnp.float32) + bnk_ref[...]

    @pl.when(i > NA)
    def _():
        qn = jnp.dot(x_ref[...], wnq_ref[...],
                     preferred_element_type=jnp.float32) + bnq_ref[...]
        s = jax.lax.dot_general(
            qn, kn_s[...], _DN_RT,
            preferred_element_type=jnp.float32) * _INV_SCALE
        m = jnp.max(s, axis=-1, keepdims=True)
        e = jnp.exp(s - m)
        w = e / jnp.sum(e, axis=-1, keepdims=True)
        h_att = (h_ref[...] * w).astype(jnp.bfloat16)
        agg = jnp.dot(h_att, att_s[...].astype(jnp.bfloat16),
                      preferred_element_type=jnp.float32)
        o_ref[...] = jnp.dot(agg, wt_ref[...],
                             preferred_element_type=jnp.float32) + bt_ref[...]


@jax.jit
def kernel(X, H_norm, Wq, bq, Wk, bk, Wv, bv, Wnq, bnq, Wnk, bnk, Wt, bt):
    f32 = jnp.float32
    bq2, bk2, bv2 = bq.reshape(1, HID), bk.reshape(1, HID), bv.reshape(1, HID)
    bnk2 = bnk.reshape(1, HID)
    bnq2, bt2 = bnq.reshape(1, HID), bt.reshape(1, F_OUT)

    full = lambda shape: pl.BlockSpec(shape, lambda *_: tuple(0 for _s in shape))

    def row_idx(i):
        return jnp.where(i < NA, i, jnp.maximum(i - (NA + 1), 0))

    out = pl.pallas_call(
        _fused_kernel,
        grid=(STEPS,),
        in_specs=[
            pl.BlockSpec((TN, F_IN), lambda i: (row_idx(i), 0)),
            pl.BlockSpec((TN, E), lambda i: (row_idx(i), 0)),
            full((F_IN, HID)), full((1, HID)),
            full((F_IN, HID)), full((1, HID)),
            full((F_IN, HID)), full((1, HID)),
            full((HID, HID)), full((1, HID)),
            full((F_IN, HID)), full((1, HID)),
            full((HID, F_OUT)), full((1, F_OUT)),
        ],
        out_specs=pl.BlockSpec((TN, F_OUT), lambda i: (row_idx(i), 0)),
        out_shape=jax.ShapeDtypeStruct((N, F_OUT), f32),
        scratch_shapes=[
            pltpu.VMEM((F_IN, E), f32),
            pltpu.VMEM((E, HID), f32),
            pltpu.VMEM((E, HID), f32),
        ],
        compiler_params=pltpu.CompilerParams(
            dimension_semantics=("arbitrary",)),
    )(X, H_norm, Wq, bq2, Wk, bk2, Wv, bv2, Wnk, bnk2, Wnq, bnq2, Wt, bt2)

    return out
